# Initial kernel scaffold; baseline (speedup 1.0000x reference)
import jax
import jax.numpy as jnp
from jax.experimental import pallas as pl


def kernel(x, Wg, bg, Wer, ber, W1, b1, W2, b2, Wo, bo, gamma, beta):
    raise NotImplementedError("write your pallas kernel here")


# speedup vs baseline: 1.0111x; 1.0111x over previous
import functools

import jax
import jax.numpy as jnp
from jax.experimental import pallas as pl
from jax.experimental.pallas import tpu as pltpu
from jax.experimental.pallas import tpu_sc as plsc

S, D, H, OUTD = 2048, 768, 3072, 768
G, EG = 4, 4
E = G * EG
BT = 128
NBCAP = (2 * S) // BT + E
PCAP = NBCAP * BT

_BF = jnp.bfloat16


def _router_body(xT_ref, wgT_ref, bg_ref, werT_ref, ber_ref, eid_ref, w_ref):
    xT = xT_ref[...]
    gl = jax.lax.dot_general(wgT_ref[...], xT, (((1,), (0,)), ((), ())),
                             preferred_element_type=jnp.float32) \
        + bg_ref[...]
    ridx = jax.lax.broadcasted_iota(jnp.int32, (G, S), 0)
    big = jnp.int32(G + 1)

    m = jnp.max(gl, axis=0, keepdims=True)
    egl = jnp.exp(gl - m)
    gp = egl / jnp.sum(egl, axis=0, keepdims=True)
    v1 = jnp.max(gp, axis=0, keepdims=True)
    i1 = jnp.min(jnp.where(gp == v1, ridx, big), axis=0, keepdims=True)
    gp2 = jnp.where(ridx == i1, -1.0, gp)
    v2 = jnp.max(gp2, axis=0, keepdims=True)
    i2 = jnp.min(jnp.where(gp2 == v2, ridx, big), axis=0, keepdims=True)

    ew = []
    ei = []
    for g in range(G):
        el = jax.lax.dot_general(werT_ref[g], xT, (((1,), (0,)), ((), ())),
                                 preferred_element_type=jnp.float32) \
            + ber_ref[g]
        mg = jnp.max(el, axis=0, keepdims=True)
        ei.append(jnp.min(jnp.where(el == mg, ridx, big), axis=0, keepdims=True))
        ew.append(1.0 / jnp.sum(jnp.exp(el - mg), axis=0, keepdims=True))

    rows_eid, rows_w = [], []
    for gsel, gwk in ((i1, v1), (i2, v2)):
        ew_sel = jnp.zeros((1, S), jnp.float32)
        ei_sel = jnp.zeros((1, S), jnp.int32)
        for g in range(G):
            hit = gsel == g
            ew_sel = jnp.where(hit, ew[g], ew_sel)
            ei_sel = jnp.where(hit, ei[g], ei_sel)
        rows_eid.append(gsel * EG + ei_sel)
        rows_w.append(gwk * ew_sel)
    eid_ref[...] = jnp.concatenate(rows_eid, axis=0)
    w_ref[...] = jnp.concatenate(rows_w, axis=0)


def _route(xT, Wg, bg, Wer, ber):
    wgT = Wg.T.astype(_BF)
    bg2 = bg.reshape(G, 1)
    werT = Wer.transpose(0, 2, 1).astype(_BF)
    ber3 = ber.reshape(G, EG, 1)
    return pl.pallas_call(
        _router_body,
        out_shape=(jax.ShapeDtypeStruct((2, S), jnp.int32),
                   jax.ShapeDtypeStruct((2, S), jnp.float32)),
    )(xT.astype(_BF), wgT, bg2, werT, ber3)


def _gather_rows(table, idx):
    n = idx.shape[0]
    dm = table.shape[1]
    hdm = dm // 2
    win = 128
    table2 = table.reshape(2 * table.shape[0], hdm)
    idx2 = jnp.stack([2 * idx, 2 * idx + 1], axis=-1).reshape(1, 2 * n)
    mesh = plsc.VectorSubcoreMesh(core_axis_name="c", subcore_axis_name="s")

    @functools.partial(pl.kernel,
                       out_type=jax.ShapeDtypeStruct((2 * n, hdm), table.dtype),
                       mesh=mesh)
    def k(x_hbm, i_hbm, o_hbm):
        def body(i_vmem, o_vmem):
            pltpu.sync_copy(x_hbm.at[i_vmem.at[0]], o_vmem)

        pltpu.emit_pipeline(
            body,
            grid=(2 * n // win,),
            in_specs=[pl.BlockSpec((1, win), lambda i: (0, i))],
            out_specs=[pl.BlockSpec((win, hdm), lambda i: (i, 0))],
            core_axis_name=("c", "s"),
            dimension_semantics=(pltpu.PARALLEL,),
        )(i_hbm, o_hbm)

    return k(table2, idx2).reshape(n, dm)


def _ffn_body(be_ref, nu_ref, xs_ref, w1_ref, b1_ref, w2_ref, b2_ref, out_ref):
    b = pl.program_id(0)

    @pl.when(b < nu_ref[0])
    def _():
        xv = xs_ref[...].astype(_BF)
        h = jnp.dot(xv, w1_ref[0],
                    preferred_element_type=jnp.float32) + b1_ref[0]
        h = jax.nn.gelu(h).astype(_BF)
        out_ref[...] = jnp.dot(h, w2_ref[0],
                               preferred_element_type=jnp.float32) + b2_ref[0]


def _ffn(xs, block_expert, nb_used, W1, b1, W2, b2):
    w1r = W1.reshape(E, D, H).astype(_BF)
    b1r = b1.reshape(E, 1, H)
    w2r = W2.reshape(E, H, OUTD).astype(_BF)
    b2r = b2.reshape(E, 1, OUTD)
    grid_spec = pltpu.PrefetchScalarGridSpec(
        num_scalar_prefetch=2,
        grid=(NBCAP,),
        in_specs=[
            pl.BlockSpec((BT, D), lambda b, be, nu: (b, 0)),
            pl.BlockSpec((1, D, H), lambda b, be, nu: (be[b], 0, 0)),
            pl.BlockSpec((1, 1, H), lambda b, be, nu: (be[b], 0, 0)),
            pl.BlockSpec((1, H, OUTD), lambda b, be, nu: (be[b], 0, 0)),
            pl.BlockSpec((1, 1, OUTD), lambda b, be, nu: (be[b], 0, 0)),
        ],
        out_specs=pl.BlockSpec((BT, OUTD), lambda b, be, nu: (b, 0)),
    )
    return pl.pallas_call(
        _ffn_body,
        grid_spec=grid_spec,
        out_shape=jax.ShapeDtypeStruct((PCAP, OUTD), jnp.float32),
    )(block_expert, nb_used, xs, w1r, b1r, w2r, b2r)


def _combine_body(yg_ref, w0_ref, w1_ref, wo_ref, bo_ref, gam_ref, bet_ref,
                  out_ref):
    comb = (w0_ref[...] * yg_ref[0:S, :] + w1_ref[...] * yg_ref[S:2 * S, :])
    z = jnp.dot(comb.astype(_BF), wo_ref[...],
                preferred_element_type=jnp.float32) + bo_ref[...]
    mu = jnp.mean(z, axis=-1, keepdims=True)
    var = jnp.mean((z - mu) ** 2, axis=-1, keepdims=True)
    out_ref[...] = (z - mu) * jax.lax.rsqrt(var + 1e-5) * gam_ref[...] \
        + bet_ref[...]


def _combine(yg, w0c, w1c, Wo, bo, gamma, beta):
    return pl.pallas_call(
        _combine_body,
        out_shape=jax.ShapeDtypeStruct((S, OUTD), jnp.float32),
    )(yg, w0c, w1c, Wo.astype(_BF), bo.reshape(1, OUTD),
      gamma.reshape(1, OUTD), beta.reshape(1, OUTD))


def kernel(x, Wg, bg, Wer, ber, W1, b1, W2, b2, Wo, bo, gamma, beta):
    x2 = x.reshape(S, D)
    eid, w = _route(x2.T, Wg, bg, Wer, ber)

    eid_flat = eid.reshape(-1)
    order = jnp.argsort(eid_flat, stable=True).astype(jnp.int32)
    sorted_eid = eid_flat[order]
    counts = jnp.bincount(eid_flat, length=E)
    offs = jnp.cumsum(counts) - counts
    pc = ((counts + BT - 1) // BT) * BT
    pstart = jnp.cumsum(pc) - pc
    j = jnp.arange(2 * S, dtype=jnp.int32)
    ppos = (pstart[sorted_eid] + j - offs[sorted_eid]).astype(jnp.int32)
    tok_sorted = (order % S).astype(jnp.int32)
    tok_padded = jnp.zeros((PCAP,), jnp.int32).at[ppos].set(tok_sorted)
    pos = jnp.zeros((2 * S,), jnp.int32).at[order].set(ppos)
    block_expert = (jnp.searchsorted(pstart // BT, jnp.arange(NBCAP),
                                     side="right") - 1).astype(jnp.int32)
    nb_used = ((pstart[E - 1] + pc[E - 1]) // BT).astype(jnp.int32).reshape(1)

    xs = _gather_rows(x2, tok_padded)
    ys = _ffn(xs, block_expert, nb_used, W1, b1, W2, b2)
    yg = _gather_rows(ys, pos)
    wt = w.T
    out = _combine(yg, wt[:, 0:1], wt[:, 1:2], Wo, bo, gamma, beta)
    return out.reshape(1, S, OUTD)


# speedup vs baseline: 1.2362x; 1.2226x over previous
import functools

import jax
import jax.numpy as jnp
from jax.experimental import pallas as pl
from jax.experimental.pallas import tpu as pltpu
from jax.experimental.pallas import tpu_sc as plsc

S, D, H, OUTD = 2048, 768, 3072, 768
G, EG = 4, 4
E = G * EG
BT = 128
NBCAP = (2 * S) // BT + E
PCAP = NBCAP * BT

_BF = jnp.bfloat16


def _router_body(xT_ref, wgT_ref, bg_ref, werT_ref, ber_ref, eid_ref, w_ref):
    xT = xT_ref[...]
    gl = jax.lax.dot_general(wgT_ref[...], xT, (((1,), (0,)), ((), ())),
                             preferred_element_type=jnp.float32) \
        + bg_ref[...]
    ridx = jax.lax.broadcasted_iota(jnp.int32, (G, S), 0)
    big = jnp.int32(G + 1)

    m = jnp.max(gl, axis=0, keepdims=True)
    egl = jnp.exp(gl - m)
    gp = egl / jnp.sum(egl, axis=0, keepdims=True)
    v1 = jnp.max(gp, axis=0, keepdims=True)
    i1 = jnp.min(jnp.where(gp == v1, ridx, big), axis=0, keepdims=True)
    gp2 = jnp.where(ridx == i1, -1.0, gp)
    v2 = jnp.max(gp2, axis=0, keepdims=True)
    i2 = jnp.min(jnp.where(gp2 == v2, ridx, big), axis=0, keepdims=True)

    ew = []
    ei = []
    for g in range(G):
        el = jax.lax.dot_general(werT_ref[g], xT, (((1,), (0,)), ((), ())),
                                 preferred_element_type=jnp.float32) \
            + ber_ref[g]
        mg = jnp.max(el, axis=0, keepdims=True)
        ei.append(jnp.min(jnp.where(el == mg, ridx, big), axis=0, keepdims=True))
        ew.append(1.0 / jnp.sum(jnp.exp(el - mg), axis=0, keepdims=True))

    rows_eid, rows_w = [], []
    for gsel, gwk in ((i1, v1), (i2, v2)):
        ew_sel = jnp.zeros((1, S), jnp.float32)
        ei_sel = jnp.zeros((1, S), jnp.int32)
        for g in range(G):
            hit = gsel == g
            ew_sel = jnp.where(hit, ew[g], ew_sel)
            ei_sel = jnp.where(hit, ei[g], ei_sel)
        rows_eid.append(gsel * EG + ei_sel)
        rows_w.append(gwk * ew_sel)
    eid_ref[...] = jnp.concatenate(rows_eid, axis=0)
    w_ref[...] = jnp.concatenate(rows_w, axis=0)


def _route(xT, Wg, bg, Wer, ber):
    wgT = Wg.T.astype(_BF)
    bg2 = bg.reshape(G, 1)
    werT = Wer.transpose(0, 2, 1).astype(_BF)
    ber3 = ber.reshape(G, EG, 1)
    return pl.pallas_call(
        _router_body,
        out_shape=(jax.ShapeDtypeStruct((2, S), jnp.int32),
                   jax.ShapeDtypeStruct((2, S), jnp.float32)),
    )(xT.astype(_BF), wgT, bg2, werT, ber3)


def _gather_rows(table, idx):
    n = idx.shape[0]
    dm = table.shape[1]
    hdm = dm // 2
    win = 128
    table2 = table.reshape(2 * table.shape[0], hdm)
    idx2 = jnp.stack([2 * idx, 2 * idx + 1], axis=-1).reshape(1, 2 * n)
    mesh = plsc.VectorSubcoreMesh(core_axis_name="c", subcore_axis_name="s")

    @functools.partial(pl.kernel,
                       out_type=jax.ShapeDtypeStruct((2 * n, hdm), table.dtype),
                       mesh=mesh)
    def k(x_hbm, i_hbm, o_hbm):
        def body(i_vmem, o_vmem):
            pltpu.sync_copy(x_hbm.at[i_vmem.at[0]], o_vmem)

        pltpu.emit_pipeline(
            body,
            grid=(2 * n // win,),
            in_specs=[pl.BlockSpec((1, win), lambda i: (0, i))],
            out_specs=[pl.BlockSpec((win, hdm), lambda i: (i, 0))],
            core_axis_name=("c", "s"),
            dimension_semantics=(pltpu.PARALLEL,),
        )(i_hbm, o_hbm)

    return k(table2, idx2).reshape(n, dm)


def _ffn_body(be_ref, nu_ref, tok_ref, rw_ref, x_ref, w1_ref, b1_ref, w2_ref,
              b2_ref, out_ref):
    b = pl.program_id(0)

    @pl.when(b < nu_ref[0])
    def _():
        tid = tok_ref[0]
        lanes = jax.lax.broadcasted_iota(jnp.int32, (BT, S), 1)
        oh = (lanes == tid).astype(_BF)
        xv = jnp.dot(oh, x_ref[...],
                     preferred_element_type=jnp.float32).astype(_BF)
        h = jnp.dot(xv, w1_ref[0],
                    preferred_element_type=jnp.float32) + b1_ref[0]
        h = jax.nn.gelu(h).astype(_BF)
        y = jnp.dot(h, w2_ref[0],
                    preferred_element_type=jnp.float32) + b2_ref[0]
        out_ref[...] = y * rw_ref[0]


def _ffn(x_bf, tok3, rw3, block_expert, nb_used, W1, b1, W2, b2):
    w1r = W1.reshape(E, D, H).astype(_BF)
    b1r = b1.reshape(E, 1, H)
    w2r = W2.reshape(E, H, OUTD).astype(_BF)
    b2r = b2.reshape(E, 1, OUTD)
    grid_spec = pltpu.PrefetchScalarGridSpec(
        num_scalar_prefetch=2,
        grid=(NBCAP,),
        in_specs=[
            pl.BlockSpec((1, BT, 1), lambda b, be, nu: (b, 0, 0)),
            pl.BlockSpec((1, BT, 1), lambda b, be, nu: (b, 0, 0)),
            pl.BlockSpec((S, D), lambda b, be, nu: (0, 0)),
            pl.BlockSpec((1, D, H), lambda b, be, nu: (be[b], 0, 0)),
            pl.BlockSpec((1, 1, H), lambda b, be, nu: (be[b], 0, 0)),
            pl.BlockSpec((1, H, OUTD), lambda b, be, nu: (be[b], 0, 0)),
            pl.BlockSpec((1, 1, OUTD), lambda b, be, nu: (be[b], 0, 0)),
        ],
        out_specs=pl.BlockSpec((BT, OUTD), lambda b, be, nu: (b, 0)),
    )
    return pl.pallas_call(
        _ffn_body,
        grid_spec=grid_spec,
        out_shape=jax.ShapeDtypeStruct((PCAP, OUTD), jnp.float32),
    )(block_expert, nb_used, tok3, rw3, x_bf, w1r, b1r, w2r, b2r)


def _combine_body(yg_ref, wo_ref, bo_ref, gam_ref, bet_ref, out_ref):
    comb = yg_ref[0:S, :] + yg_ref[S:2 * S, :]
    z = jnp.dot(comb.astype(_BF), wo_ref[...],
                preferred_element_type=jnp.float32) + bo_ref[...]
    mu = jnp.mean(z, axis=-1, keepdims=True)
    var = jnp.mean((z - mu) ** 2, axis=-1, keepdims=True)
    out_ref[...] = (z - mu) * jax.lax.rsqrt(var + 1e-5) * gam_ref[...] \
        + bet_ref[...]


def _combine(yg, Wo, bo, gamma, beta):
    return pl.pallas_call(
        _combine_body,
        out_shape=jax.ShapeDtypeStruct((S, OUTD), jnp.float32),
    )(yg, Wo.astype(_BF), bo.reshape(1, OUTD),
      gamma.reshape(1, OUTD), beta.reshape(1, OUTD))


def kernel(x, Wg, bg, Wer, ber, W1, b1, W2, b2, Wo, bo, gamma, beta):
    x2 = x.reshape(S, D)
    eid, w = _route(x2.T, Wg, bg, Wer, ber)

    eid_flat = eid.reshape(-1)
    order = jnp.argsort(eid_flat, stable=True).astype(jnp.int32)
    sorted_eid = eid_flat[order]
    counts = jnp.bincount(eid_flat, length=E)
    offs = jnp.cumsum(counts) - counts
    pc = ((counts + BT - 1) // BT) * BT
    pstart = jnp.cumsum(pc) - pc
    j = jnp.arange(2 * S, dtype=jnp.int32)
    ppos = (pstart[sorted_eid] + j - offs[sorted_eid]).astype(jnp.int32)
    tok_sorted = (order % S).astype(jnp.int32)
    tok_padded = jnp.zeros((PCAP,), jnp.int32).at[ppos].set(tok_sorted)
    rw_padded = jnp.zeros((PCAP,), jnp.float32).at[ppos].set(w.reshape(-1)[order])
    pos = jnp.zeros((2 * S,), jnp.int32).at[order].set(ppos)
    block_expert = (jnp.searchsorted(pstart // BT, jnp.arange(NBCAP),
                                     side="right") - 1).astype(jnp.int32)
    nb_used = ((pstart[E - 1] + pc[E - 1]) // BT).astype(jnp.int32).reshape(1)

    ys = _ffn(x2.astype(_BF), tok_padded.reshape(NBCAP, BT, 1),
              rw_padded.reshape(NBCAP, BT, 1), block_expert, nb_used,
              W1, b1, W2, b2)
    yg = _gather_rows(ys, pos)
    out = _combine(yg, Wo, bo, gamma, beta)
    return out.reshape(1, S, OUTD)


# speedup vs baseline: 1.2537x; 1.0141x over previous
import functools

import jax
import jax.numpy as jnp
from jax.experimental import pallas as pl
from jax.experimental.pallas import tpu as pltpu
from jax.experimental.pallas import tpu_sc as plsc

S, D, H, OUTD = 2048, 768, 3072, 768
G, EG = 4, 4
E = G * EG
BT = 256
NBCAP = (2 * S) // BT + E
PCAP = NBCAP * BT

_BF = jnp.bfloat16


def _router_body(xT_ref, wgT_ref, bg_ref, werT_ref, ber_ref, eid_ref, w_ref):
    xT = xT_ref[...]
    gl = jax.lax.dot_general(wgT_ref[...], xT, (((1,), (0,)), ((), ())),
                             preferred_element_type=jnp.float32) \
        + bg_ref[...]
    ridx = jax.lax.broadcasted_iota(jnp.int32, (G, S), 0)
    big = jnp.int32(G + 1)

    m = jnp.max(gl, axis=0, keepdims=True)
    egl = jnp.exp(gl - m)
    gp = egl / jnp.sum(egl, axis=0, keepdims=True)
    v1 = jnp.max(gp, axis=0, keepdims=True)
    i1 = jnp.min(jnp.where(gp == v1, ridx, big), axis=0, keepdims=True)
    gp2 = jnp.where(ridx == i1, -1.0, gp)
    v2 = jnp.max(gp2, axis=0, keepdims=True)
    i2 = jnp.min(jnp.where(gp2 == v2, ridx, big), axis=0, keepdims=True)

    ew = []
    ei = []
    for g in range(G):
        el = jax.lax.dot_general(werT_ref[g], xT, (((1,), (0,)), ((), ())),
                                 preferred_element_type=jnp.float32) \
            + ber_ref[g]
        mg = jnp.max(el, axis=0, keepdims=True)
        ei.append(jnp.min(jnp.where(el == mg, ridx, big), axis=0, keepdims=True))
        ew.append(1.0 / jnp.sum(jnp.exp(el - mg), axis=0, keepdims=True))

    rows_eid, rows_w = [], []
    for gsel, gwk in ((i1, v1), (i2, v2)):
        ew_sel = jnp.zeros((1, S), jnp.float32)
        ei_sel = jnp.zeros((1, S), jnp.int32)
        for g in range(G):
            hit = gsel == g
            ew_sel = jnp.where(hit, ew[g], ew_sel)
            ei_sel = jnp.where(hit, ei[g], ei_sel)
        rows_eid.append(gsel * EG + ei_sel)
        rows_w.append(gwk * ew_sel)
    eid_ref[...] = jnp.concatenate(rows_eid, axis=0)
    w_ref[...] = jnp.concatenate(rows_w, axis=0)


def _route(xT, Wg, bg, Wer, ber):
    wgT = Wg.T.astype(_BF)
    bg2 = bg.reshape(G, 1)
    werT = Wer.transpose(0, 2, 1).astype(_BF)
    ber3 = ber.reshape(G, EG, 1)
    return pl.pallas_call(
        _router_body,
        out_shape=(jax.ShapeDtypeStruct((2, S), jnp.int32),
                   jax.ShapeDtypeStruct((2, S), jnp.float32)),
    )(xT.astype(_BF), wgT, bg2, werT, ber3)


def _gather_rows(table, idx):
    n = idx.shape[0]
    dm = table.shape[1]
    hdm = dm // 2
    win = 128
    table2 = table.reshape(2 * table.shape[0], hdm)
    idx2 = jnp.stack([2 * idx, 2 * idx + 1], axis=-1).reshape(1, 2 * n)
    mesh = plsc.VectorSubcoreMesh(core_axis_name="c", subcore_axis_name="s")

    @functools.partial(pl.kernel,
                       out_type=jax.ShapeDtypeStruct((2 * n, hdm), table.dtype),
                       mesh=mesh)
    def k(x_hbm, i_hbm, o_hbm):
        def body(i_vmem, o_vmem):
            pltpu.sync_copy(x_hbm.at[i_vmem.at[0]], o_vmem)

        pltpu.emit_pipeline(
            body,
            grid=(2 * n // win,),
            in_specs=[pl.BlockSpec((1, win), lambda i: (0, i))],
            out_specs=[pl.BlockSpec((win, hdm), lambda i: (i, 0))],
            core_axis_name=("c", "s"),
            dimension_semantics=(pltpu.PARALLEL,),
        )(i_hbm, o_hbm)

    return k(table2, idx2).reshape(n, dm)


def _ffn_body(be_ref, nu_ref, tok_ref, rw_ref, x_ref, w1_ref, b1_ref, w2_ref,
              b2_ref, out_ref):
    b = pl.program_id(0)

    @pl.when(b < nu_ref[0])
    def _():
        tid = tok_ref[0]
        lanes = jax.lax.broadcasted_iota(jnp.int32, (BT, S), 1)
        oh = (lanes == tid).astype(_BF)
        xv = jnp.dot(oh, x_ref[...],
                     preferred_element_type=jnp.float32).astype(_BF)
        h = jnp.dot(xv, w1_ref[0],
                    preferred_element_type=jnp.float32) + b1_ref[0]
        h = jax.nn.gelu(h.astype(_BF))
        y = jnp.dot(h, w2_ref[0],
                    preferred_element_type=jnp.float32) + b2_ref[0]
        out_ref[...] = y * rw_ref[0]


def _ffn(x_bf, tok3, rw3, block_expert, nb_used, W1, b1, W2, b2):
    w1r = W1.reshape(E, D, H).astype(_BF)
    b1r = b1.reshape(E, 1, H)
    w2r = W2.reshape(E, H, OUTD).astype(_BF)
    b2r = b2.reshape(E, 1, OUTD)
    grid_spec = pltpu.PrefetchScalarGridSpec(
        num_scalar_prefetch=2,
        grid=(NBCAP,),
        in_specs=[
            pl.BlockSpec((1, BT, 1), lambda b, be, nu: (b, 0, 0)),
            pl.BlockSpec((1, BT, 1), lambda b, be, nu: (b, 0, 0)),
            pl.BlockSpec((S, D), lambda b, be, nu: (0, 0)),
            pl.BlockSpec((1, D, H), lambda b, be, nu: (be[b], 0, 0)),
            pl.BlockSpec((1, 1, H), lambda b, be, nu: (be[b], 0, 0)),
            pl.BlockSpec((1, H, OUTD), lambda b, be, nu: (be[b], 0, 0)),
            pl.BlockSpec((1, 1, OUTD), lambda b, be, nu: (be[b], 0, 0)),
        ],
        out_specs=pl.BlockSpec((BT, OUTD), lambda b, be, nu: (b, 0)),
    )
    return pl.pallas_call(
        _ffn_body,
        grid_spec=grid_spec,
        out_shape=jax.ShapeDtypeStruct((PCAP, OUTD), jnp.float32),
    )(block_expert, nb_used, tok3, rw3, x_bf, w1r, b1r, w2r, b2r)


def _combine_body(yg_ref, wo_ref, bo_ref, gam_ref, bet_ref, out_ref):
    comb = yg_ref[0:S, :] + yg_ref[S:2 * S, :]
    z = jnp.dot(comb.astype(_BF), wo_ref[...],
                preferred_element_type=jnp.float32) + bo_ref[...]
    mu = jnp.mean(z, axis=-1, keepdims=True)
    var = jnp.mean((z - mu) ** 2, axis=-1, keepdims=True)
    out_ref[...] = (z - mu) * jax.lax.rsqrt(var + 1e-5) * gam_ref[...] \
        + bet_ref[...]


def _combine(yg, Wo, bo, gamma, beta):
    return pl.pallas_call(
        _combine_body,
        out_shape=jax.ShapeDtypeStruct((S, OUTD), jnp.float32),
    )(yg, Wo.astype(_BF), bo.reshape(1, OUTD),
      gamma.reshape(1, OUTD), beta.reshape(1, OUTD))


def kernel(x, Wg, bg, Wer, ber, W1, b1, W2, b2, Wo, bo, gamma, beta):
    x2 = x.reshape(S, D)
    eid, w = _route(x2.T, Wg, bg, Wer, ber)

    eid_flat = eid.reshape(-1)
    order = jnp.argsort(eid_flat, stable=True).astype(jnp.int32)
    sorted_eid = eid_flat[order]
    counts = jnp.bincount(eid_flat, length=E)
    offs = jnp.cumsum(counts) - counts
    pc = ((counts + BT - 1) // BT) * BT
    pstart = jnp.cumsum(pc) - pc
    j = jnp.arange(2 * S, dtype=jnp.int32)
    ppos = (pstart[sorted_eid] + j - offs[sorted_eid]).astype(jnp.int32)
    tok_sorted = (order % S).astype(jnp.int32)
    tok_padded = jnp.zeros((PCAP,), jnp.int32).at[ppos].set(tok_sorted)
    rw_padded = jnp.zeros((PCAP,), jnp.float32).at[ppos].set(w.reshape(-1)[order])
    pos = jnp.zeros((2 * S,), jnp.int32).at[order].set(ppos)
    block_expert = (jnp.searchsorted(pstart // BT, jnp.arange(NBCAP),
                                     side="right") - 1).astype(jnp.int32)
    nb_used = ((pstart[E - 1] + pc[E - 1]) // BT).astype(jnp.int32).reshape(1)

    ys = _ffn(x2.astype(_BF), tok_padded.reshape(NBCAP, BT, 1),
              rw_padded.reshape(NBCAP, BT, 1), block_expert, nb_used,
              W1, b1, W2, b2)
    yg = _gather_rows(ys, pos)
    out = _combine(yg, Wo, bo, gamma, beta)
    return out.reshape(1, S, OUTD)
